# Initial kernel scaffold; baseline (speedup 1.0000x reference)
import jax
import jax.numpy as jnp
from jax.experimental import pallas as pl


def kernel(full_output, indices):
    raise NotImplementedError("write your pallas kernel here")


# speedup vs baseline: 74.8288x; 74.8288x over previous
import functools

import jax
import jax.numpy as jnp
from jax import lax
from jax.experimental import pallas as pl
from jax.experimental.pallas import tpu as pltpu
from jax.experimental.pallas import tpu_sc as plsc

OUT_D = 64
NB_ACT = 26
BATCH = 16384

NC = 2
NS = 16
L = 16
NW = NC * NS
B_PER_W = BATCH // NW


def _gather_kernel(table_hbm, idx_hbm, out_hbm, idx_v, rows_v, sem):
    wid = lax.axis_index("s") * NC + lax.axis_index("c")
    base = wid * B_PER_W

    pltpu.sync_copy(idx_hbm.at[pl.ds(base, B_PER_W)], idx_v)

    lane = lax.iota(jnp.int32, L)

    def body(i, _):
        off = i * L
        v = idx_v[pl.ds(off, L)]
        idx_v[pl.ds(off, L)] = (base + off + lane) * NB_ACT + v
        return 0

    lax.fori_loop(0, B_PER_W // L, body, 0)

    pltpu.async_copy(table_hbm.at[idx_v], rows_v, sem).wait()
    pltpu.sync_copy(rows_v, out_hbm.at[pl.ds(base, B_PER_W)])


@jax.jit
def _gather(table, idx):
    mesh = plsc.VectorSubcoreMesh(core_axis_name="c", subcore_axis_name="s")
    return pl.kernel(
        _gather_kernel,
        out_type=jax.ShapeDtypeStruct((BATCH, OUT_D), jnp.float32),
        mesh=mesh,
        scratch_types=[
            pltpu.VMEM((B_PER_W,), jnp.int32),
            pltpu.VMEM((B_PER_W, OUT_D), jnp.float32),
            pltpu.SemaphoreType.DMA,
        ],
        compiler_params=pltpu.CompilerParams(use_tc_tiling_on_sc=False),
    )(table, idx)


def kernel(full_output, indices):
    table = full_output.reshape(BATCH * NB_ACT, OUT_D)
    idx = indices.reshape(BATCH).astype(jnp.int32)
    return _gather(table, idx)


# speedup vs baseline: 129.1086x; 1.7254x over previous
import functools

import jax
import jax.numpy as jnp
from jax import lax
from jax.experimental import pallas as pl
from jax.experimental.pallas import tpu as pltpu
from jax.experimental.pallas import tpu_sc as plsc

OUT_D = 64
NB_ACT = 26
BATCH = 16384
WIDTH = OUT_D * NB_ACT

NC = 2
NS = 16
L = 16
NW = NC * NS
B_PER_W = BATCH // NW
G = 32
NCH = B_PER_W // G
NBUF = 2


def _slice_kernel(full_hbm, idx_hbm, out_hbm, idx_v, chunk_v, out_v, sems):
    wid = lax.axis_index("s") * NC + lax.axis_index("c")
    base = wid * B_PER_W

    pltpu.sync_copy(idx_hbm.at[pl.ds(base, B_PER_W)], idx_v)

    lane = lax.iota(jnp.int32, L)

    def chunk_copy(g, buf):
        return pltpu.make_async_copy(
            full_hbm.at[pl.ds(base + g * G, G)],
            chunk_v.at[buf],
            sems.at[buf],
        )

    chunk_copy(0, 0).start()
    chunk_copy(1, 1).start()

    def body(g, _):
        buf = lax.rem(g, 2)
        chunk_copy(g, buf).wait()
        bufl = jnp.full((L,), buf, jnp.int32)
        for grp in range(G // L):
            row = grp * L + lane
            col0 = idx_v[pl.ds(g * G + grp * L, L)] * OUT_D
            for c in range(OUT_D):
                vals = plsc.load_gather(chunk_v, [bufl, row, col0 + c])
                plsc.store_scatter(
                    out_v, [bufl, row, jnp.full((L,), c, jnp.int32)], vals
                )
        @pl.when(g + NBUF < NCH)
        def _():
            chunk_copy(g + NBUF, buf).start()

        pltpu.sync_copy(
            out_v.at[buf], out_hbm.at[pl.ds(base + g * G, G)]
        )
        return 0

    lax.fori_loop(0, NCH, body, 0)


@jax.jit
def _run(full_output, idx):
    mesh = plsc.VectorSubcoreMesh(core_axis_name="c", subcore_axis_name="s")
    return pl.kernel(
        _slice_kernel,
        out_type=jax.ShapeDtypeStruct((BATCH, OUT_D), jnp.float32),
        mesh=mesh,
        scratch_types=[
            pltpu.VMEM((B_PER_W,), jnp.int32),
            pltpu.VMEM((NBUF, G, WIDTH), jnp.float32),
            pltpu.VMEM((NBUF, G, OUT_D), jnp.float32),
            pltpu.SemaphoreType.DMA((NBUF,)),
        ],
        compiler_params=pltpu.CompilerParams(needs_layout_passes=False),
    )(full_output, idx)


def kernel(full_output, indices):
    idx = indices.reshape(BATCH).astype(jnp.int32)
    return _run(full_output, idx)


# speedup vs baseline: 142.2454x; 1.1017x over previous
import functools

import jax
import jax.numpy as jnp
from jax import lax
from jax.experimental import pallas as pl
from jax.experimental.pallas import tpu as pltpu
from jax.experimental.pallas import tpu_sc as plsc

OUT_D = 64
NB_ACT = 26
BATCH = 16384
WIDTH = OUT_D * NB_ACT

NC = 2
NS = 16
L = 16
NW = NC * NS
B_PER_W = BATCH // NW
G = 16
NCH = B_PER_W // G
NBUF = 4


def _slice_kernel(full_hbm, idx_hbm, out_hbm, idx_v, chunk_v, out_v,
                  in_sems, out_sems):
    wid = lax.axis_index("s") * NC + lax.axis_index("c")
    base = wid * B_PER_W

    pltpu.sync_copy(idx_hbm.at[pl.ds(base, B_PER_W)], idx_v)

    lane = lax.iota(jnp.int32, L)

    def in_copy(g, buf):
        return pltpu.make_async_copy(
            full_hbm.at[pl.ds(base + g * G, G)],
            chunk_v.at[buf],
            in_sems.at[buf],
        )

    def out_copy(g, buf):
        return pltpu.make_async_copy(
            out_v.at[buf],
            out_hbm.at[pl.ds(base + g * G, G)],
            out_sems.at[buf],
        )

    for b in range(NBUF):
        in_copy(b, b).start()

    def body(g, _):
        buf = lax.rem(g, NBUF)
        in_copy(g, buf).wait()
        @pl.when(g >= NBUF)
        def _():
            out_copy(g - NBUF, buf).wait()

        bufl = jnp.full((L,), buf, jnp.int32)
        for grp in range(G // L):
            row = grp * L + lane
            col0 = idx_v[pl.ds(g * G + grp * L, L)] * OUT_D
            for c in range(OUT_D):
                vals = plsc.load_gather(chunk_v, [bufl, row, col0 + c])
                plsc.store_scatter(
                    out_v, [bufl, row, jnp.full((L,), c, jnp.int32)], vals
                )
        out_copy(g, buf).start()

        @pl.when(g + NBUF < NCH)
        def _():
            in_copy(g + NBUF, buf).start()

        return 0

    lax.fori_loop(0, NCH, body, 0)

    for b in range(NBUF):
        out_copy(NCH - NBUF + b, lax.rem(NCH - NBUF + b, NBUF)).wait()


@jax.jit
def _run(full_output, idx):
    mesh = plsc.VectorSubcoreMesh(core_axis_name="c", subcore_axis_name="s")
    return pl.kernel(
        _slice_kernel,
        out_type=jax.ShapeDtypeStruct((BATCH, OUT_D), jnp.float32),
        mesh=mesh,
        scratch_types=[
            pltpu.VMEM((B_PER_W,), jnp.int32),
            pltpu.VMEM((NBUF, G, WIDTH), jnp.float32),
            pltpu.VMEM((NBUF, G, OUT_D), jnp.float32),
            pltpu.SemaphoreType.DMA((NBUF,)),
            pltpu.SemaphoreType.DMA((NBUF,)),
        ],
        compiler_params=pltpu.CompilerParams(needs_layout_passes=False),
    )(full_output, idx)


def kernel(full_output, indices):
    idx = indices.reshape(BATCH).astype(jnp.int32)
    return _run(full_output, idx)


# speedup vs baseline: 144.0262x; 1.0125x over previous
import functools

import jax
import jax.numpy as jnp
from jax import lax
from jax.experimental import pallas as pl
from jax.experimental.pallas import tpu as pltpu
from jax.experimental.pallas import tpu_sc as plsc

OUT_D = 64
NB_ACT = 26
BATCH = 16384
WIDTH = OUT_D * NB_ACT

NC = 2
NS = 16
L = 16
NW = NC * NS
B_PER_W = BATCH // NW
G = 16
NG = B_PER_W // G
NBUF = 4


def _slice_kernel(full_hbm, idx_hbm, out_hbm, idx_v, land_v, out_v,
                  in_sems, out_sems):
    wid = lax.axis_index("s") * NC + lax.axis_index("c")
    base = wid * B_PER_W

    pltpu.sync_copy(idx_hbm.at[pl.ds(base, B_PER_W)], idx_v)

    lane = lax.iota(jnp.int32, L)

    def start_group(g, buf):
        j16 = lax.div(idx_v[pl.ds(g * G, L)], 2) * 128
        for l in range(L):
            col = pl.multiple_of(j16[l], 128)
            pltpu.make_async_copy(
                full_hbm.at[pl.ds(base + g * G + (l & ~7), 8),
                            pl.ds(col, 128)],
                land_v.at[buf, l],
                in_sems.at[buf],
            ).start()

    def wait_group(buf):
        for l in range(L):
            pltpu.make_async_copy(
                full_hbm.at[pl.ds(0, 8), pl.ds(0, 128)],
                land_v.at[buf, l],
                in_sems.at[buf],
            ).wait()

    def out_copy(g, buf):
        return pltpu.make_async_copy(
            out_v.at[buf],
            out_hbm.at[pl.ds(base + g * G, G)],
            out_sems.at[buf],
        )

    for b in range(NBUF):
        start_group(b, b)

    sub = lax.rem(lane, 8)

    def body(g, _):
        buf = lax.rem(g, NBUF)
        wait_group(buf)

        @pl.when(g >= NBUF)
        def _():
            out_copy(g - NBUF, buf).wait()

        v16 = idx_v[pl.ds(g * G, L)]
        h16 = lax.rem(v16, 2) * OUT_D
        bufl = jnp.full((L,), buf, jnp.int32)
        for c in range(OUT_D):
            vals = plsc.load_gather(land_v, [bufl, lane, sub, h16 + c])
            plsc.store_scatter(
                out_v, [bufl, lane, jnp.full((L,), c, jnp.int32)], vals
            )
        out_copy(g, buf).start()

        @pl.when(g + NBUF < NG)
        def _():
            start_group(g + NBUF, buf)

        return 0

    lax.fori_loop(0, NG, body, 0)

    for b in range(NBUF):
        out_copy(NG - NBUF + b, lax.rem(NG - NBUF + b, NBUF)).wait()


@jax.jit
def _run(full_output, idx):
    mesh = plsc.VectorSubcoreMesh(core_axis_name="c", subcore_axis_name="s")
    return pl.kernel(
        _slice_kernel,
        out_type=jax.ShapeDtypeStruct((BATCH, OUT_D), jnp.float32),
        mesh=mesh,
        scratch_types=[
            pltpu.VMEM((B_PER_W,), jnp.int32),
            pltpu.VMEM((NBUF, G, 8, 128), jnp.float32),
            pltpu.VMEM((NBUF, G, OUT_D), jnp.float32),
            pltpu.SemaphoreType.DMA((NBUF,)),
            pltpu.SemaphoreType.DMA((NBUF,)),
        ],
        compiler_params=pltpu.CompilerParams(needs_layout_passes=False),
    )(full_output, idx)


def kernel(full_output, indices):
    idx = indices.reshape(BATCH).astype(jnp.int32)
    return _run(full_output, idx)
